# baseline (speedup 1.0000x reference)
import functools

import jax
import jax.numpy as jnp
from jax import lax
from jax.experimental import pallas as pl
from jax.experimental.pallas import tpu as pltpu
from jax.experimental.pallas import tpu_sc as plsc

F = 26
V1 = 100001
B = 16384
D = 32
L = 16

NW = 32
TOTAL = B * F
PER_W = TOTAL // NW
CHUNK = 1024
N_CHUNK = PER_W // CHUNK
ROWS_PER_DMA = 128
NDMA = CHUNK // ROWS_PER_DMA
N_SLICES = CHUNK // L
SL_PER_ROW = ROWS_PER_DMA // L


@functools.partial(
    pl.kernel,
    out_type=jax.ShapeDtypeStruct((TOTAL, D), jnp.float32),
    mesh=plsc.VectorSubcoreMesh(core_axis_name="c", subcore_axis_name="s"),
    compiler_params=pltpu.CompilerParams(use_tc_tiling_on_sc=False),
    scratch_types=[
        pltpu.VMEM((CHUNK,), jnp.int32),
        pltpu.VMEM((NDMA, ROWS_PER_DMA), jnp.int32),
        pltpu.VMEM((CHUNK, D), jnp.float32),
        pltpu.SemaphoreType.DMA,
    ],
)
def _lookup(idx_hbm, table_hbm, out_hbm, idx_v, fidx_v, rows_v, sem):
    wid = lax.axis_index("s") * 2 + lax.axis_index("c")
    base = wid * PER_W

    def chunk_body(g, carry):
        off = base + g * CHUNK
        pltpu.sync_copy(idx_hbm.at[pl.ds(off, CHUNK)], idx_v)
        for p in range(N_SLICES):
            pos = off + p * L + lax.iota(jnp.int32, 16)
            fld = lax.rem(pos, F)
            raw = idx_v[pl.ds(p * L, L)]
            flat = jnp.maximum(raw, 0) + fld * V1
            fidx_v[p // SL_PER_ROW, pl.ds((p % SL_PER_ROW) * L, L)] = flat
        cps = []
        for j in range(NDMA):
            cps.append(
                pltpu.async_copy(
                    table_hbm.at[fidx_v.at[j]],
                    rows_v.at[pl.ds(j * ROWS_PER_DMA, ROWS_PER_DMA)],
                    sem,
                )
            )
        for cp in cps:
            cp.wait()
        pltpu.sync_copy(rows_v, out_hbm.at[pl.ds(off, CHUNK)])
        return carry

    lax.fori_loop(0, N_CHUNK, chunk_body, 0)


def kernel(x_cat, tables):
    idx_flat = x_cat.astype(jnp.int32).reshape(TOTAL)
    table_flat = tables.reshape(F * V1, D)
    out = _lookup(idx_flat, table_flat)
    return out.reshape(B, F, D)


# speedup vs baseline: 38.8981x; 38.8981x over previous
import functools

import jax
import jax.numpy as jnp
from jax import lax
from jax.experimental import pallas as pl
from jax.experimental.pallas import tpu as pltpu
from jax.experimental.pallas import tpu_sc as plsc

F = 26
V = 100001
B = 16384
D = 32
L = 16

NW = 32
ROWS = F * D
ROWS_PER_W = ROWS // NW
HALF = B // 2
SL_PER_STEP = 8
STEPS = HALF // (L * SL_PER_STEP)


@functools.partial(
    pl.kernel,
    out_type=jax.ShapeDtypeStruct((F, D, B), jnp.float32),
    mesh=plsc.VectorSubcoreMesh(core_axis_name="c", subcore_axis_name="s"),
    compiler_params=pltpu.CompilerParams(needs_layout_passes=False),
    scratch_types=[
        pltpu.VMEM((V,), jnp.float32),
        pltpu.VMEM((HALF,), jnp.int32),
        pltpu.VMEM((HALF,), jnp.float32),
    ],
)
def _lookup(idx_hbm, tab_hbm, out_hbm, row_v, idx_v, outb_v):
    wid = lax.axis_index("s") * 2 + lax.axis_index("c")

    def row_body(r, carry):
        fd = wid * ROWS_PER_W + r
        f = fd // D
        d = fd - f * D
        pltpu.sync_copy(tab_hbm.at[f, d], row_v)
        for h in range(2):
            pltpu.sync_copy(idx_hbm.at[f, pl.ds(h * HALF, HALF)], idx_v)

            def step(j, c):
                base = j * (L * SL_PER_STEP)
                for s in range(SL_PER_STEP):
                    o = base + s * L
                    v = jnp.maximum(idx_v[pl.ds(o, L)], 0)
                    outb_v[pl.ds(o, L)] = plsc.load_gather(row_v, [v])
                return c

            lax.fori_loop(0, STEPS, step, 0)
            pltpu.sync_copy(outb_v, out_hbm.at[f, d, pl.ds(h * HALF, HALF)])
        return carry

    lax.fori_loop(0, ROWS_PER_W, row_body, 0)


def kernel(x_cat, tables):
    idx_t = jnp.transpose(x_cat.astype(jnp.int32))
    tab_t = jnp.transpose(tables, (0, 2, 1))
    out_t = _lookup(idx_t, tab_t)
    return jnp.transpose(out_t, (2, 0, 1))


# speedup vs baseline: 46.7485x; 1.2018x over previous
import functools

import jax
import jax.numpy as jnp
from jax import lax
from jax.experimental import pallas as pl
from jax.experimental.pallas import tpu as pltpu
from jax.experimental.pallas import tpu_sc as plsc

F = 26
V = 100001
B = 16384
D = 32
L = 16

NW = 32
ROWS = F * D
ROWS_PER_W = ROWS // NW
HALF = B // 2
SL_PER_STEP = 8
STEPS = HALF // (L * SL_PER_STEP)


@functools.partial(
    pl.kernel,
    out_type=jax.ShapeDtypeStruct((F, D, B), jnp.float32),
    mesh=plsc.VectorSubcoreMesh(core_axis_name="c", subcore_axis_name="s"),
    compiler_params=pltpu.CompilerParams(needs_layout_passes=False),
    scratch_types=[
        pltpu.VMEM((V,), jnp.float32),
        pltpu.VMEM((B,), jnp.int32),
        pltpu.VMEM((HALF,), jnp.float32),
    ],
)
def _lookup(idx_hbm, tab_hbm, out_hbm, row_v, idx_v, outb_v):
    wid = lax.axis_index("s") * 2 + lax.axis_index("c")

    def row_body(r, prev_f):
        fd = wid * ROWS_PER_W + r
        f = fd // D
        d = fd - f * D

        @pl.when(f != prev_f)
        def _():
            pltpu.sync_copy(idx_hbm.at[f], idx_v)

        pltpu.sync_copy(tab_hbm.at[f, d], row_v)
        for h in range(2):

            def step(j, c):
                base = h * HALF + j * (L * SL_PER_STEP)
                for s in range(SL_PER_STEP):
                    o = base + s * L
                    v = jnp.maximum(idx_v[pl.ds(o, L)], 0)
                    outb_v[pl.ds(o - h * HALF, L)] = plsc.load_gather(row_v, [v])
                return c

            lax.fori_loop(0, STEPS, step, 0)
            pltpu.sync_copy(outb_v, out_hbm.at[f, d, pl.ds(h * HALF, HALF)])
        return f

    lax.fori_loop(0, ROWS_PER_W, row_body, -1)


def kernel(x_cat, tables):
    idx_t = jnp.transpose(x_cat.astype(jnp.int32))
    tab_t = jnp.transpose(tables, (0, 2, 1))
    out_t = _lookup(idx_t, tab_t)
    return jnp.transpose(out_t, (2, 0, 1))
